# baseline (speedup 1.0000x reference)
import jax
import jax.numpy as jnp
from jax.experimental import pallas as pl
from jax.experimental.pallas import tpu as pltpu

D_MODEL = 768
NUM_EXPERTS = 64
BT = 2048
CH = 2048


def _router_kernel(x_ref, w_ref, b_ref, probs_ref, idx_ref):
    w = w_ref[:]
    bias = b_ref[:]
    for c in range(BT // CH):
        logits = jax.lax.dot_general(
            w, x_ref[pl.ds(c * CH, CH), :],
            dimension_numbers=(((1,), (1,)), ((), ())),
            preferred_element_type=jnp.float32,
        ) + bias
        subl = jax.lax.broadcasted_iota(jnp.int32, logits.shape, 0)

        v0 = jnp.max(logits, axis=0, keepdims=True)
        i0 = jnp.min(jnp.where(logits == v0, subl, NUM_EXPERTS), axis=0,
                     keepdims=True)
        masked = jnp.where(subl == i0, -jnp.inf, logits)
        v1 = jnp.max(masked, axis=0, keepdims=True)
        i1 = jnp.min(jnp.where(masked == v1, subl, NUM_EXPERTS), axis=0,
                     keepdims=True)

        e = jnp.exp(v1 - v0)
        p0 = 1.0 / (1.0 + e)
        p1 = e * p0

        probs_ref[:, pl.ds(c * CH, CH)] = jnp.concatenate([p0, p1], axis=0)
        idx_ref[:, pl.ds(c * CH, CH)] = jnp.concatenate([i0, i1], axis=0)


def kernel(x, W, b):
    n = x.shape[0]
    probs_t, idx_t = pl.pallas_call(
        _router_kernel,
        grid=(n // BT,),
        in_specs=[
            pl.BlockSpec((BT, D_MODEL), lambda i: (i, 0)),
            pl.BlockSpec((NUM_EXPERTS, D_MODEL), lambda i: (0, 0)),
            pl.BlockSpec((NUM_EXPERTS, 1), lambda i: (0, 0)),
        ],
        out_specs=[
            pl.BlockSpec((2, BT), lambda i: (0, i)),
            pl.BlockSpec((2, BT), lambda i: (0, i)),
        ],
        out_shape=[
            jax.ShapeDtypeStruct((2, n), jnp.float32),
            jax.ShapeDtypeStruct((2, n), jnp.int32),
        ],
        compiler_params=pltpu.CompilerParams(
            dimension_semantics=("arbitrary",),
        ),
    )(x, W, b.reshape(NUM_EXPERTS, 1))
    return (probs_t.T, idx_t.T)


# speedup vs baseline: 1.0619x; 1.0619x over previous
import jax
import jax.numpy as jnp
from jax.experimental import pallas as pl
from jax.experimental.pallas import tpu as pltpu

D_MODEL = 768
NUM_EXPERTS = 64
BT = 4096
CH = 4096


def _router_kernel(x_ref, w_ref, b_ref, probs_ref, idx_ref):
    w = w_ref[:]
    bias = b_ref[:]
    for c in range(BT // CH):
        logits = jax.lax.dot_general(
            w, x_ref[pl.ds(c * CH, CH), :],
            dimension_numbers=(((1,), (1,)), ((), ())),
            preferred_element_type=jnp.float32,
        ) + bias
        subl = jax.lax.broadcasted_iota(jnp.int32, logits.shape, 0)

        v0 = jnp.max(logits, axis=0, keepdims=True)
        i0 = jnp.min(jnp.where(logits == v0, subl, NUM_EXPERTS), axis=0,
                     keepdims=True)
        masked = jnp.where(subl == i0, -jnp.inf, logits)
        v1 = jnp.max(masked, axis=0, keepdims=True)
        i1 = jnp.min(jnp.where(masked == v1, subl, NUM_EXPERTS), axis=0,
                     keepdims=True)

        e = jnp.exp(v1 - v0)
        p0 = 1.0 / (1.0 + e)
        p1 = e * p0

        probs_ref[:, pl.ds(c * CH, CH)] = jnp.concatenate([p0, p1], axis=0)
        idx_ref[:, pl.ds(c * CH, CH)] = jnp.concatenate([i0, i1], axis=0)


def kernel(x, W, b):
    n = x.shape[0]
    probs_t, idx_t = pl.pallas_call(
        _router_kernel,
        grid=(n // BT,),
        in_specs=[
            pl.BlockSpec((BT, D_MODEL), lambda i: (i, 0)),
            pl.BlockSpec((NUM_EXPERTS, D_MODEL), lambda i: (0, 0)),
            pl.BlockSpec((NUM_EXPERTS, 1), lambda i: (0, 0)),
        ],
        out_specs=[
            pl.BlockSpec((2, BT), lambda i: (0, i)),
            pl.BlockSpec((2, BT), lambda i: (0, i)),
        ],
        out_shape=[
            jax.ShapeDtypeStruct((2, n), jnp.float32),
            jax.ShapeDtypeStruct((2, n), jnp.int32),
        ],
        compiler_params=pltpu.CompilerParams(
            dimension_semantics=("arbitrary",),
        ),
    )(x, W, b.reshape(NUM_EXPERTS, 1))
    return (probs_t.T, idx_t.T)
